# Initial kernel scaffold; baseline (speedup 1.0000x reference)
import jax
import jax.numpy as jnp
from jax.experimental import pallas as pl


def kernel(posi_idx, cos_cached, sin_cached):
    raise NotImplementedError("write your pallas kernel here")


# speedup vs baseline: 4.7334x; 4.7334x over previous
import functools

import jax
import jax.numpy as jnp
from jax import lax
from jax.experimental import pallas as pl
from jax.experimental.pallas import tpu as pltpu
from jax.experimental.pallas import tpu_sc as plsc

HID_DIM = 128
CHUNK = 128


def _make_gather(n_idx):
    info = plsc.get_sparse_core_info()
    nc, ns = info.num_cores, info.num_subcores
    nw = nc * ns
    per_w = n_idx // nw
    n_chunks = per_w // CHUNK

    mesh = plsc.VectorSubcoreMesh(core_axis_name="c", subcore_axis_name="s")
    out_sds = jax.ShapeDtypeStruct((n_idx, HID_DIM), jnp.float32)

    @functools.partial(
        pl.kernel,
        mesh=mesh,
        out_type=(out_sds, out_sds),
        scratch_types=[
            pltpu.VMEM((n_chunks, CHUNK), jnp.int32),
            pltpu.VMEM((CHUNK, HID_DIM), jnp.float32),
            pltpu.VMEM((CHUNK, HID_DIM), jnp.float32),
            pltpu.SemaphoreType.DMA,
        ],
    )
    def gather_kernel(cos_hbm, sin_hbm, idx_hbm, cos_out, sin_out,
                      idx_v, cos_rows, sin_rows, sem):
        wid = lax.axis_index("s") * nc + lax.axis_index("c")
        row0 = wid * n_chunks
        pltpu.sync_copy(idx_hbm.at[pl.ds(row0, n_chunks)], idx_v)
        for j in range(n_chunks):
            base = (row0 + j) * CHUNK
            cp_c = pltpu.async_copy(cos_hbm.at[idx_v.at[j]], cos_rows, sem)
            cp_s = pltpu.async_copy(sin_hbm.at[idx_v.at[j]], sin_rows, sem)
            cp_c.wait()
            cp_s.wait()
            pltpu.sync_copy(cos_rows, cos_out.at[pl.ds(base, CHUNK)])
            pltpu.sync_copy(sin_rows, sin_out.at[pl.ds(base, CHUNK)])

    return gather_kernel


@jax.jit
def kernel(posi_idx, cos_cached, sin_cached):
    b, s = posi_idx.shape
    n_idx = b * s
    idx2d = posi_idx.reshape(n_idx // CHUNK, CHUNK).astype(jnp.int32)
    cos_flat, sin_flat = _make_gather(n_idx)(cos_cached, sin_cached, idx2d)
    return (cos_flat.reshape(b, s, HID_DIM), sin_flat.reshape(b, s, HID_DIM))


# speedup vs baseline: 5.1330x; 1.0844x over previous
import functools

import jax
import jax.numpy as jnp
from jax import lax
from jax.experimental import pallas as pl
from jax.experimental.pallas import tpu as pltpu
from jax.experimental.pallas import tpu_sc as plsc

HID_DIM = 128
CHUNK = 128
NBUF = 2


def _make_gather(n_idx):
    info = plsc.get_sparse_core_info()
    nc, ns = info.num_cores, info.num_subcores
    nw = nc * ns
    per_w = n_idx // nw
    n_chunks = per_w // CHUNK

    mesh = plsc.VectorSubcoreMesh(core_axis_name="c", subcore_axis_name="s")
    out_sds = jax.ShapeDtypeStruct((n_idx, HID_DIM), jnp.float32)

    @functools.partial(
        pl.kernel,
        mesh=mesh,
        out_type=(out_sds, out_sds),
        scratch_types=[
            pltpu.VMEM((n_chunks, CHUNK), jnp.int32),
            pltpu.VMEM((NBUF, CHUNK, HID_DIM), jnp.float32),
            pltpu.VMEM((NBUF, CHUNK, HID_DIM), jnp.float32),
            pltpu.SemaphoreType.DMA((NBUF,)),
            pltpu.SemaphoreType.DMA((NBUF,)),
        ],
    )
    def gather_kernel(cos_hbm, sin_hbm, idx_hbm, cos_out, sin_out,
                      idx_v, cos_rows, sin_rows, sem_in, sem_out):
        wid = lax.axis_index("s") * nc + lax.axis_index("c")
        row0 = wid * n_chunks
        pltpu.sync_copy(idx_hbm.at[pl.ds(row0, n_chunks)], idx_v)

        gathers = {}
        writes = {}

        def issue_gather(j):
            b = j % NBUF
            gathers[j] = (
                pltpu.async_copy(cos_hbm.at[idx_v.at[j]], cos_rows.at[b], sem_in.at[b]),
                pltpu.async_copy(sin_hbm.at[idx_v.at[j]], sin_rows.at[b], sem_in.at[b]),
            )

        def issue_write(j):
            b = j % NBUF
            base = (row0 + j) * CHUNK
            writes[j] = (
                pltpu.async_copy(cos_rows.at[b], cos_out.at[pl.ds(base, CHUNK)], sem_out.at[b]),
                pltpu.async_copy(sin_rows.at[b], sin_out.at[pl.ds(base, CHUNK)], sem_out.at[b]),
            )

        issue_gather(0)
        for j in range(n_chunks):
            if j + 1 < n_chunks:
                if j >= 1:
                    writes[j - 1][0].wait()
                    writes[j - 1][1].wait()
                issue_gather(j + 1)
            gathers[j][0].wait()
            gathers[j][1].wait()
            issue_write(j)
        for j in range(max(0, n_chunks - NBUF), n_chunks):
            writes[j][0].wait()
            writes[j][1].wait()

    return gather_kernel


@jax.jit
def kernel(posi_idx, cos_cached, sin_cached):
    b, s = posi_idx.shape
    n_idx = b * s
    idx2d = posi_idx.reshape(n_idx // CHUNK, CHUNK).astype(jnp.int32)
    cos_flat, sin_flat = _make_gather(n_idx)(cos_cached, sin_cached, idx2d)
    return (cos_flat.reshape(b, s, HID_DIM), sin_flat.reshape(b, s, HID_DIM))
